# baseline (speedup 1.0000x reference)
import functools

import jax
import jax.numpy as jnp
from jax import lax
from jax.experimental import pallas as pl
from jax.experimental.pallas import tpu as pltpu
from jax.experimental.pallas import tpu_sc as plsc

_C = 8
_G = 8
_B = _C * _G
_D = 500
_R = _G + _C - 1
_TOT = 2 * _B * _R
_NW = 32
_PER_W = _TOT // _NW
_DP = 512
_PW = 64

@functools.cache
def _build_sc_gather():
    mesh = plsc.VectorSubcoreMesh(core_axis_name="c", subcore_axis_name="s")

    @functools.partial(
        pl.kernel,
        mesh=mesh,
        out_type=jax.ShapeDtypeStruct((_NW, _PER_W * _D), jnp.float32),
        scratch_types=[
            pltpu.VMEM((_PW,), jnp.int32),
            pltpu.VMEM((_PW, _DP), jnp.float32),
            pltpu.VMEM((_PER_W * _D,), jnp.float32),
            pltpu.SemaphoreType.DMA,
        ],
        compiler_params=pltpu.CompilerParams(use_tc_tiling_on_sc=False),
    )
    def _sc_gather(table_hbm, gidx_hbm, out_hbm, idx_v, rows_v, packed_v, sem):
        wid = lax.axis_index("s") * 2 + lax.axis_index("c")
        pltpu.sync_copy(gidx_hbm.at[wid], idx_v)
        pltpu.async_copy(table_hbm.at[idx_v], rows_v, sem).wait()

        def body(r, _):
            dst = r * _D
            for c in range(0, _D - 16, 16):
                packed_v[pl.ds(dst + c, 16)] = rows_v[r, pl.ds(c, 16)]
            packed_v[pl.ds(dst + _D - 16, 16)] = rows_v[r, pl.ds(_D - 16, 16)]
            return ()

        lax.fori_loop(0, _PER_W, body, (), unroll=False)
        pltpu.sync_copy(packed_v, out_hbm.at[wid])

    return _sc_gather


def kernel(inputs, targets):
    anchor_class = targets.astype(jnp.int32)
    pos_src = anchor_class[:, None] * _G + jnp.arange(_G, dtype=jnp.int32)[None, :]
    idx = jnp.arange(_C - 1, dtype=jnp.int32)
    neg_cls = idx[None, :] + (idx[None, :] >= anchor_class[:, None]).astype(jnp.int32)
    j = jax.random.randint(jax.random.key(1), (_B, _C - 1), 1, _G)
    neg_src = neg_cls * _G + j.astype(jnp.int32)
    src = jnp.concatenate([pos_src, neg_src], axis=1)
    anchors = jnp.broadcast_to(
        jnp.arange(_B, dtype=jnp.int32)[:, None], (_B, _R))
    gidx = jnp.stack([anchors, src], axis=-1).reshape(_NW, _PER_W)
    gidx = jnp.pad(gidx, ((0, 0), (0, _PW - _PER_W)))
    table = jnp.pad(inputs, ((0, 0), (0, _DP - _D)))

    out = _build_sc_gather()(table, gidx)
    expanded = out.reshape(_B * _R, 2 * _D)

    labels = jnp.concatenate(
        [jnp.ones((_G,), jnp.int32), jnp.zeros((_C - 1,), jnp.int32)])
    new_targets = jnp.tile(labels, (_B,))
    return new_targets, expanded


# speedup vs baseline: 1.1102x; 1.1102x over previous
import functools

import jax
import jax.numpy as jnp
from jax import lax
from jax.experimental import pallas as pl
from jax.experimental.pallas import tpu as pltpu
from jax.experimental.pallas import tpu_sc as plsc

_C = 8
_G = 8
_B = _C * _G
_D = 500
_R = _G + _C - 1
_TOT = 2 * _B * _R
_NW = 32
_PER_W = _TOT // _NW
_DP = 512
_PW = 64

@functools.cache
def _build_sc_gather():
    mesh = plsc.VectorSubcoreMesh(core_axis_name="c", subcore_axis_name="s")

    @functools.partial(
        pl.kernel,
        mesh=mesh,
        out_type=jax.ShapeDtypeStruct((_NW, _PER_W * _D), jnp.float32),
        scratch_types=[
            pltpu.VMEM((_PW,), jnp.int32),
            pltpu.VMEM((_PW, _DP), jnp.float32),
            pltpu.VMEM((_PER_W * _D,), jnp.float32),
            pltpu.SemaphoreType.DMA,
        ],
        compiler_params=pltpu.CompilerParams(use_tc_tiling_on_sc=False),
    )
    def _sc_gather(table_hbm, gidx_hbm, out_hbm, idx_v, rows_v, packed_v, sem):
        wid = lax.axis_index("s") * 2 + lax.axis_index("c")
        pltpu.sync_copy(gidx_hbm.at[wid], idx_v)
        pltpu.async_copy(table_hbm.at[idx_v], rows_v, sem).wait()

        @plsc.parallel_loop(0, _PER_W, unroll=4)
        def _compact(r):
            dst = r * _D
            for c in range(0, _D - 16, 16):
                packed_v[pl.ds(dst + c, 16)] = rows_v[r, pl.ds(c, 16)]
            packed_v[pl.ds(dst + _D - 16, 16)] = rows_v[r, pl.ds(_D - 16, 16)]

        pltpu.sync_copy(packed_v, out_hbm.at[wid])

    return _sc_gather


def kernel(inputs, targets):
    anchor_class = targets.astype(jnp.int32)
    pos_src = anchor_class[:, None] * _G + jnp.arange(_G, dtype=jnp.int32)[None, :]
    idx = jnp.arange(_C - 1, dtype=jnp.int32)
    neg_cls = idx[None, :] + (idx[None, :] >= anchor_class[:, None]).astype(jnp.int32)
    j = jax.random.randint(jax.random.key(1), (_B, _C - 1), 1, _G)
    neg_src = neg_cls * _G + j.astype(jnp.int32)
    src = jnp.concatenate([pos_src, neg_src], axis=1)
    anchors = jnp.broadcast_to(
        jnp.arange(_B, dtype=jnp.int32)[:, None], (_B, _R))
    gidx = jnp.stack([anchors, src], axis=-1).reshape(_NW, _PER_W)
    gidx = jnp.pad(gidx, ((0, 0), (0, _PW - _PER_W)))
    table = jnp.pad(inputs, ((0, 0), (0, _DP - _D)))

    out = _build_sc_gather()(table, gidx)
    expanded = out.reshape(_B * _R, 2 * _D)

    labels = jnp.concatenate(
        [jnp.ones((_G,), jnp.int32), jnp.zeros((_C - 1,), jnp.int32)])
    new_targets = jnp.tile(labels, (_B,))
    return new_targets, expanded
